# baseline (speedup 1.0000x reference)
import functools

import jax
import jax.numpy as jnp
from jax import lax
from jax.experimental import pallas as pl
from jax.experimental.pallas import tpu as pltpu
from jax.experimental.pallas import tpu_sc as plsc

L = 16
G = 128

_PERM_DNUMS = lax.GatherDimensionNumbers(
    offset_dims=(), collapsed_slice_dims=(0,), start_index_map=(0,))


def _vperm(x, perm):
    return lax.gather(x, perm[:, None], _PERM_DNUMS, slice_sizes=(1,),
                      mode=lax.GatherScatterMode.PROMISE_IN_BOUNDS)


@jax.jit
def _gather_gps(table_flat, n):
    B = n.shape[0]
    info = plsc.get_sparse_core_info()
    num_workers = info.num_subcores
    bw = B // num_workers
    ng = 2 * bw // G
    mesh = plsc.VectorSubcoreMesh(
        core_axis_name="c", subcore_axis_name="s", num_cores=1)

    @functools.partial(
        pl.kernel,
        mesh=mesh,
        out_type=jax.ShapeDtypeStruct((2 * B,), jnp.float32),
        compiler_params=pltpu.CompilerParams(use_tc_tiling_on_sc=False),
        scratch_types=[
            pltpu.VMEM((bw,), jnp.int32),
            [pltpu.VMEM((G,), jnp.int32) for _ in range(ng)],
            [pltpu.VMEM((G,), jnp.float32) for _ in range(ng)],
            pltpu.SemaphoreType.DMA,
        ],
    )
    def k(table_hbm, idx_hbm, out_hbm, idx_v, iidx, dat, sem):
        wid = lax.axis_index("s")
        base = wid * bw
        pltpu.sync_copy(idx_hbm.at[pl.ds(base, bw)], idx_v)
        lane = lax.iota(jnp.int32, L)
        parity = lane & 1
        half = lane >> 1
        handles = []
        for g in range(ng):
            for c in range(G // L):
                blk = (g * (G // L) + c) // 2
                nblk = idx_v[pl.ds(blk * L, L)]
                nv = _vperm(nblk, half + 8 * (c % 2))
                iidx[g][pl.ds(c * L, L)] = nv * 2 + parity
            handles.append(
                pltpu.async_copy(table_hbm.at[iidx[g]], dat[g], sem))
        scale = jnp.where(parity == 0, 90.0, 180.0)
        for g in range(ng):
            handles[g].wait()
            for c in range(G // L):
                v = dat[g][pl.ds(c * L, L)]
                dat[g][pl.ds(c * L, L)] = (
                    jnp.minimum(jnp.maximum(v, -1.0), 1.0) * scale)
            pltpu.sync_copy(dat[g], out_hbm.at[pl.ds(2 * base + g * G, G)])

    return k(table_flat, n)


def kernel(img, coordinates, n):
    del img
    flat = _gather_gps(coordinates.reshape(-1), n)
    return flat.reshape(n.shape[0], 2)


# speedup vs baseline: 3.8765x; 3.8765x over previous
import functools

import jax
import jax.numpy as jnp
from jax import lax
from jax.experimental import pallas as pl
from jax.experimental.pallas import tpu as pltpu
from jax.experimental.pallas import tpu_sc as plsc

L = 16
BLK = 128


@jax.jit
def _gather_gps(table_view, n):
    B = n.shape[0]
    info = plsc.get_sparse_core_info()
    num_workers = info.num_subcores
    bw = B // num_workers
    nb = bw // BLK
    mesh = plsc.VectorSubcoreMesh(
        core_axis_name="c", subcore_axis_name="s", num_cores=1)

    @functools.partial(
        pl.kernel,
        mesh=mesh,
        out_type=jax.ShapeDtypeStruct((2 * B,), jnp.float32),
        compiler_params=pltpu.CompilerParams(use_tc_tiling_on_sc=False),
        scratch_types=[
            pltpu.VMEM((bw,), jnp.int32),
            [pltpu.VMEM((BLK,), jnp.int32) for _ in range(nb)],
            [pltpu.VMEM((BLK,), jnp.int32) for _ in range(nb)],
            [pltpu.VMEM((BLK,), jnp.float32) for _ in range(nb)],
            [pltpu.VMEM((BLK,), jnp.float32) for _ in range(nb)],
            pltpu.SemaphoreType.DMA,
        ],
    )
    def k(table_hbm, idx_hbm, out_hbm, idx_v, lati, loni, latd, lond, sem):
        wid = lax.axis_index("s")
        base = wid * bw
        pltpu.sync_copy(idx_hbm.at[pl.ds(base, bw)], idx_v)
        handles = []
        for kb in range(nb):
            for c in range(BLK // L):
                nc = idx_v[pl.ds(kb * BLK + c * L, L)]
                addr = lax.shift_left(
                    lax.shift_right_logical(nc, 7), 8) + (nc & 127)
                lati[kb][pl.ds(c * L, L)] = addr
                loni[kb][pl.ds(c * L, L)] = addr + BLK
            handles.append(
                pltpu.async_copy(table_hbm.at[lati[kb]], latd[kb], sem))
            handles.append(
                pltpu.async_copy(table_hbm.at[loni[kb]], lond[kb], sem))
        for kb in range(nb):
            handles[2 * kb].wait()
            handles[2 * kb + 1].wait()
            for c in range(BLK // L):
                la = latd[kb][pl.ds(c * L, L)]
                latd[kb][pl.ds(c * L, L)] = (
                    jnp.minimum(jnp.maximum(la, -1.0), 1.0) * 90.0)
                lo = lond[kb][pl.ds(c * L, L)]
                lond[kb][pl.ds(c * L, L)] = (
                    jnp.minimum(jnp.maximum(lo, -1.0), 1.0) * 180.0)
            blk_off = (base + kb * BLK) * 2
            pltpu.sync_copy(latd[kb], out_hbm.at[pl.ds(blk_off, BLK)])
            pltpu.sync_copy(lond[kb], out_hbm.at[pl.ds(blk_off + BLK, BLK)])

    return k(table_view, n)


def kernel(img, coordinates, n):
    del img
    N = coordinates.shape[0]
    B = n.shape[0]
    npad = (-N) % BLK
    tpad = jnp.pad(coordinates, ((0, npad), (0, 0)))
    nblk = (N + npad) // BLK
    tview = tpad.reshape(nblk, BLK, 2).transpose(0, 2, 1).reshape(-1)
    flat = _gather_gps(tview, n)
    return flat.reshape(B // BLK, 2, BLK).transpose(0, 2, 1).reshape(B, 2)


# speedup vs baseline: 3.9039x; 1.0071x over previous
import functools

import jax
import jax.numpy as jnp
from jax import lax
from jax.experimental import pallas as pl
from jax.experimental.pallas import tpu as pltpu
from jax.experimental.pallas import tpu_sc as plsc

L = 16
BLK = 128


@jax.jit
def _gather_gps(table_view, n):
    B = n.shape[0]
    info = plsc.get_sparse_core_info()
    num_workers = info.num_subcores
    bw = B // num_workers
    nb = bw // BLK
    mesh = plsc.VectorSubcoreMesh(
        core_axis_name="c", subcore_axis_name="s", num_cores=1)

    @functools.partial(
        pl.kernel,
        mesh=mesh,
        out_type=jax.ShapeDtypeStruct((2 * B,), jnp.float32),
        compiler_params=pltpu.CompilerParams(use_tc_tiling_on_sc=False),
        scratch_types=[
            pltpu.VMEM((bw,), jnp.int32),
            [pltpu.VMEM((BLK,), jnp.int32) for _ in range(nb)],
            [pltpu.VMEM((BLK,), jnp.int32) for _ in range(nb)],
            [pltpu.VMEM((BLK,), jnp.float32) for _ in range(nb)],
            [pltpu.VMEM((BLK,), jnp.float32) for _ in range(nb)],
            pltpu.SemaphoreType.DMA,
            pltpu.SemaphoreType.DMA,
        ],
    )
    def k(table_hbm, idx_hbm, out_hbm, idx_v, lati, loni, latd, lond, sem,
          osem):
        wid = lax.axis_index("s")
        base = wid * bw
        pltpu.sync_copy(idx_hbm.at[pl.ds(base, bw)], idx_v)
        handles = []
        for kb in range(nb):
            for c in range(BLK // L):
                nc = idx_v[pl.ds(kb * BLK + c * L, L)]
                addr = lax.shift_left(
                    lax.shift_right_logical(nc, 7), 8) + (nc & 127)
                lati[kb][pl.ds(c * L, L)] = addr
                loni[kb][pl.ds(c * L, L)] = addr + BLK
            handles.append(
                pltpu.async_copy(table_hbm.at[lati[kb]], latd[kb], sem))
            handles.append(
                pltpu.async_copy(table_hbm.at[loni[kb]], lond[kb], sem))
        out_handles = []
        for kb in range(nb):
            handles[2 * kb].wait()
            handles[2 * kb + 1].wait()
            for c in range(BLK // L):
                la = latd[kb][pl.ds(c * L, L)]
                latd[kb][pl.ds(c * L, L)] = (
                    jnp.minimum(jnp.maximum(la, -1.0), 1.0) * 90.0)
                lo = lond[kb][pl.ds(c * L, L)]
                lond[kb][pl.ds(c * L, L)] = (
                    jnp.minimum(jnp.maximum(lo, -1.0), 1.0) * 180.0)
            blk_off = (base + kb * BLK) * 2
            out_handles.append(pltpu.async_copy(
                latd[kb], out_hbm.at[pl.ds(blk_off, BLK)], osem))
            out_handles.append(pltpu.async_copy(
                lond[kb], out_hbm.at[pl.ds(blk_off + BLK, BLK)], osem))
        for h in out_handles:
            h.wait()

    return k(table_view, n)


def kernel(img, coordinates, n):
    del img
    N = coordinates.shape[0]
    B = n.shape[0]
    npad = (-N) % BLK
    tpad = jnp.pad(coordinates, ((0, npad), (0, 0)))
    nblk = (N + npad) // BLK
    tview = tpad.reshape(nblk, BLK, 2).transpose(0, 2, 1).reshape(-1)
    flat = _gather_gps(tview, n)
    return flat.reshape(B // BLK, 2, BLK).transpose(0, 2, 1).reshape(B, 2)
